# Initial kernel scaffold; baseline (speedup 1.0000x reference)
import jax
import jax.numpy as jnp
from jax.experimental import pallas as pl


def kernel(x, alpha, levels):
    raise NotImplementedError("write your pallas kernel here")


# speedup vs baseline: 11347.6802x; 11347.6802x over previous
import functools

import jax
import jax.numpy as jnp
from jax import lax
from jax.experimental import pallas as pl
from jax.experimental.pallas import tpu as pltpu
from jax.experimental.pallas import tpu_sc as plsc

SHIFT = 15
OFF = (95 << 23) >> SHIFT
ROWS = 72
NK = ROWS * 128
TINY = 2.0 ** -32
CHUNK = 16384
NWORK = 32


def _lut_build_body(alpha_ref, pos_ref, cut_ref, lo_ref, hi_ref):
    cid = (lax.broadcasted_iota(jnp.int32, (ROWS, 128), 0) * 128
           + lax.broadcasted_iota(jnp.int32, (ROWS, 128), 1))
    blo = (cid + OFF) << SHIFT
    bhi = blo | ((1 << SHIFT) - 1)
    vlo = lax.bitcast_convert_type(blo, jnp.float32)
    vhi = lax.bitcast_convert_type(bhi, jnp.float32)
    n_mids = pos_ref.shape[0] - 1

    def body(j, accs):
        alo, ahi = accs
        p0 = pos_ref[j]
        p1 = pos_ref[j + 1]
        m = (p0 + p1) * 0.5
        d = p1 - p0
        alo = alo + jnp.where(vlo > m, d, 0.0)
        ahi = ahi + jnp.where(vhi > m, d, 0.0)
        return (alo, ahi)

    z = jnp.zeros((ROWS, 128), jnp.float32)
    alo, ahi = lax.fori_loop(0, n_mids, body, (z, z))
    a = jnp.abs(alpha_ref[0]) + 1e-8
    cut_ref[...] = (alo + ahi) * 0.5
    lo_ref[...] = alo * a
    hi_ref[...] = ahi * a


def _build_luts(alpha, pos):
    return pl.pallas_call(
        _lut_build_body,
        out_shape=[jax.ShapeDtypeStruct((ROWS, 128), jnp.float32)] * 3,
        in_specs=[pl.BlockSpec(memory_space=pltpu.SMEM),
                  pl.BlockSpec(memory_space=pltpu.SMEM)],
    )(alpha, pos)


@functools.cache
def _make_sc_quantize(n):
    per = n // NWORK
    n_chunks = per // CHUNK
    mesh = plsc.VectorSubcoreMesh(core_axis_name="c", subcore_axis_name="s")

    @functools.partial(
        pl.kernel,
        mesh=mesh,
        compiler_params=pltpu.CompilerParams(needs_layout_passes=False),
        out_type=jax.ShapeDtypeStruct((n,), jnp.float32),
        scratch_types=[
            pltpu.VMEM((NK,), jnp.float32),
            pltpu.VMEM((NK,), jnp.float32),
            pltpu.VMEM((NK,), jnp.float32),
            pltpu.VMEM((16,), jnp.float32),
            pltpu.VMEM((2, CHUNK), jnp.float32),
            pltpu.VMEM((2, CHUNK), jnp.float32),
            pltpu.SemaphoreType.DMA,
            pltpu.SemaphoreType.DMA,
            pltpu.SemaphoreType.DMA,
            pltpu.SemaphoreType.DMA,
        ],
    )
    def quantize(x_hbm, a16_hbm, cut_hbm, lo_hbm, hi_hbm, out_hbm,
                 cut_v, lo_v, hi_v, a_v, in_v, out_v,
                 ld_sem0, ld_sem1, st_sem0, st_sem1):
        wid = lax.axis_index("s") * 2 + lax.axis_index("c")
        pltpu.sync_copy(cut_hbm, cut_v)
        pltpu.sync_copy(lo_hbm, lo_v)
        pltpu.sync_copy(hi_hbm, hi_v)
        pltpu.sync_copy(a16_hbm, a_v)
        av = a_v[...]
        inv = 1.0 / (jnp.abs(av) + 1e-8)
        base0 = wid * per
        ld_sems = (ld_sem0, ld_sem1)
        st_sems = (st_sem0, st_sem1)

        def start_load(g, b):
            pltpu.async_copy(
                x_hbm.at[pl.ds(base0 + g * CHUNK, CHUNK)], in_v.at[b],
                ld_sems[b])

        def compute(b):
            @pl.loop(0, CHUNK // 16)
            def _(i):
                sl = pl.ds(i * 16, 16)
                xv = in_v[b, sl]
                bi = plsc.bitcast(xv, jnp.int32)
                sign = lax.bitwise_and(bi, jnp.int32(-2147483648))
                ab = lax.bitwise_and(bi, jnp.int32(2147483647))
                t = plsc.bitcast(ab, jnp.float32) * inv
                t = jnp.minimum(t, 1.0)
                t = jnp.maximum(t, TINY)
                key = (lax.shift_right_logical(plsc.bitcast(t, jnp.int32),
                                               SHIFT) - OFF)
                c = plsc.load_gather(cut_v, [key])
                l = plsc.load_gather(lo_v, [key])
                h = plsc.load_gather(hi_v, [key])
                sel = jnp.where(t > c, h, l)
                ob = lax.bitwise_or(plsc.bitcast(sel, jnp.int32), sign)
                out_v[b, sl] = plsc.bitcast(ob, jnp.float32)

        def start_store(g, b):
            pltpu.async_copy(
                out_v.at[b], out_hbm.at[pl.ds(base0 + g * CHUNK, CHUNK)],
                st_sems[b])

        start_load(0, 0)

        @pl.loop(0, n_chunks, step=2)
        def _(g):
            for b in range(2):
                gg = g + b
                nb = 1 - b

                @pl.when(gg + 1 < n_chunks)
                def _():
                    start_load(gg + 1, nb)

                pltpu.make_async_copy(
                    x_hbm.at[pl.ds(base0 + gg * CHUNK, CHUNK)], in_v.at[b],
                    ld_sems[b]).wait()

                @pl.when(gg >= 2)
                def _():
                    pltpu.make_async_copy(
                        out_v.at[b],
                        out_hbm.at[pl.ds(base0 + (gg - 2) * CHUNK, CHUNK)],
                        st_sems[b]).wait()

                compute(b)
                start_store(gg, b)

        for b in range(2):
            g_last = n_chunks - 2 + b
            pltpu.make_async_copy(
                out_v.at[b],
                out_hbm.at[pl.ds(base0 + g_last * CHUNK, CHUNK)],
                st_sems[b]).wait()

    return quantize


def kernel(x, alpha, levels):
    L = levels.shape[0]
    pos = levels[(L - 1) // 2:]
    alpha = alpha.astype(jnp.float32)
    cut, lo, hi = _build_luts(alpha, pos.astype(jnp.float32))
    a16 = jnp.broadcast_to(alpha, (16,))
    n = x.size
    assert n % (NWORK * CHUNK) == 0, n
    out = _make_sc_quantize(n)(
        x.reshape(n), a16, cut.reshape(NK), lo.reshape(NK), hi.reshape(NK))
    return out.reshape(x.shape)
